# baseline (speedup 1.0000x reference)
import functools

import jax
import jax.numpy as jnp
from jax import lax
from jax.experimental import pallas as pl
from jax.experimental.pallas import tpu as pltpu
from jax.experimental.pallas import tpu_sc as plsc

N = 10000
E = 320000
D = 128

NC = 2
NS = 16
NW = NC * NS
C = 128
NCHUNK = 80
EPW = C * NCHUNK
E_PAD = NW * EPW
WB = 80
RPT = 640
RPT_LAST_CHUNKS = (N - (NS - 1) * RPT) // WB
RPT_CHUNKS = RPT // WB


def _mm_body(f_ref, wt_ref, b_ref, o_ref):
    o_ref[...] = (
        jnp.dot(f_ref[...], wt_ref[...], preferred_element_type=jnp.float32)
        + b_ref[...]
    )


def _add_body(p_ref, o_ref):
    o_ref[...] = p_ref[0] + p_ref[1]


def _bcast_lane(v, lane):
    return lax.gather(
        v, jnp.full((16, 1), lane, jnp.int32),
        lax.GatherDimensionNumbers(
            offset_dims=(), collapsed_slice_dims=(0,), start_index_map=(0,)),
        (1,), mode=lax.GatherScatterMode.PROMISE_IN_BOUNDS)


def _sc_spmm_body(support_hbm, rows_hbm, cols_hbm, vals_hbm, out_hbm,
                  cbuf, rbufa, rbufb, vbufa, vbufb, bufa, bufb, acc,
                  gsema, gsemb, esema, esemb, ssem):
    cid = lax.axis_index("c")
    sid = lax.axis_index("s")
    wid = cid * NS + sid
    zero16 = jnp.zeros((16,), jnp.float32)

    pltpu.sync_copy(cols_hbm.at[wid], cbuf)

    def zrow(g, carry):
        for j in range(D // 16):
            bufa[g, pl.ds(j * 16, 16)] = zero16
        return carry
    lax.fori_loop(0, C, zrow, 0)

    row0 = sid * RPT
    nwb = jnp.where(sid == NS - 1, RPT_LAST_CHUNKS, RPT_CHUNKS)

    def zacc(k, carry):
        pltpu.sync_copy(bufa.at[pl.ds(0, WB)], acc.at[pl.ds(row0 + k * WB, WB)])
        return carry
    lax.fori_loop(0, nwb, zacc, 0)

    plsc.subcore_barrier()

    def start_all(ci, buf, rbuf, vbuf, gsem, esem):
        pltpu.async_copy(support_hbm.at[cbuf.at[ci]], buf, gsem)
        pltpu.async_copy(rows_hbm.at[wid, ci], rbuf, esem)
        pltpu.async_copy(vals_hbm.at[wid, ci], vbuf, esem)

    def wait_all(ci, buf, rbuf, vbuf, gsem, esem):
        pltpu.make_async_copy(support_hbm.at[cbuf.at[ci]], buf, gsem).wait()
        pltpu.make_async_copy(rows_hbm.at[wid, ci], rbuf, esem).wait()
        pltpu.make_async_copy(vals_hbm.at[wid, ci], vbuf, esem).wait()

    def scale(buf, vbuf):
        def body(g, carry):
            vv = vbuf[pl.ds(g * 16, 16)]
            for i2 in range(16):
                r = g * 16 + i2
                s = _bcast_lane(vv, i2)
                for j in range(D // 16):
                    buf[r, pl.ds(j * 16, 16)] = buf[r, pl.ds(j * 16, 16)] * s
            return carry
        lax.fori_loop(0, C // 16, body, 0)

    def scatter(buf, rbuf):
        pltpu.async_copy(buf, acc.at[rbuf], ssem, add=True).wait()

    start_all(0, bufa, rbufa, vbufa, gsema, esema)

    def pair(k, carry):
        ci = 2 * k
        wait_all(ci, bufa, rbufa, vbufa, gsema, esema)
        start_all(ci + 1, bufb, rbufb, vbufb, gsemb, esemb)
        scale(bufa, vbufa)
        scatter(bufa, rbufa)
        wait_all(ci + 1, bufb, rbufb, vbufb, gsemb, esemb)

        @pl.when(k < NCHUNK // 2 - 1)
        def _():
            start_all(ci + 2, bufa, rbufa, vbufa, gsema, esema)
        scale(bufb, vbufb)
        scatter(bufb, rbufb)
        return carry
    lax.fori_loop(0, NCHUNK // 2, pair, 0)

    plsc.subcore_barrier()

    def wb(k, carry):
        sl = pl.ds(row0 + k * WB, WB)
        pltpu.sync_copy(acc.at[sl], bufa.at[pl.ds(0, WB)])
        pltpu.sync_copy(bufa.at[pl.ds(0, WB)], out_hbm.at[cid, sl])
        return carry
    lax.fori_loop(0, nwb, wb, 0)


_sc_spmm = functools.partial(
    pl.kernel,
    out_type=jax.ShapeDtypeStruct((NC, N, D), jnp.float32),
    mesh=plsc.VectorSubcoreMesh(
        core_axis_name="c", subcore_axis_name="s",
        num_cores=NC, num_subcores=NS),
    scratch_types=[
        pltpu.VMEM((NCHUNK, C), jnp.int32),
        pltpu.VMEM((C,), jnp.int32),
        pltpu.VMEM((C,), jnp.int32),
        pltpu.VMEM((C,), jnp.float32),
        pltpu.VMEM((C,), jnp.float32),
        pltpu.VMEM((C, D), jnp.float32),
        pltpu.VMEM((C, D), jnp.float32),
        pltpu.VMEM_SHARED((N, D), jnp.float32),
        pltpu.SemaphoreType.DMA,
        pltpu.SemaphoreType.DMA,
        pltpu.SemaphoreType.DMA,
        pltpu.SemaphoreType.DMA,
        pltpu.SemaphoreType.DMA,
    ],
)(_sc_spmm_body)


def kernel(laplacian_indices, laplacian_values, features, W, b):
    pad = E_PAD - E
    idx_p = jnp.pad(laplacian_indices, ((0, 0), (0, pad)))
    vals_p = jnp.pad(laplacian_values, (0, pad))
    rows3 = idx_p[0].reshape(NW, NCHUNK, C)
    cols3 = idx_p[1].reshape(NW, NCHUNK, C)
    vals3 = vals_p.reshape(NW, NCHUNK, C)
    wt = W.T
    b2 = b.reshape(1, D)

    support = pl.pallas_call(
        _mm_body,
        grid=(10,),
        in_specs=[
            pl.BlockSpec((N // 10, D), lambda i: (i, 0)),
            pl.BlockSpec((D, D), lambda i: (0, 0)),
            pl.BlockSpec((1, D), lambda i: (0, 0)),
        ],
        out_specs=pl.BlockSpec((N // 10, D), lambda i: (i, 0)),
        out_shape=jax.ShapeDtypeStruct((N, D), jnp.float32),
    )(features, wt, b2)

    partials = _sc_spmm(support, rows3, cols3, vals3)

    out = pl.pallas_call(
        _add_body,
        grid=(10,),
        in_specs=[pl.BlockSpec((NC, N // 10, D), lambda i: (0, i, 0))],
        out_specs=pl.BlockSpec((N // 10, D), lambda i: (i, 0)),
        out_shape=jax.ShapeDtypeStruct((N, D), jnp.float32),
    )(partials)
    return out


# speedup vs baseline: 2.7336x; 2.7336x over previous
import functools

import jax
import jax.numpy as jnp
from jax import lax
from jax.experimental import pallas as pl
from jax.experimental.pallas import tpu as pltpu
from jax.experimental.pallas import tpu_sc as plsc

N = 10000
E = 320000
D = 128

NC = 2
NS = 16
NW = NC * NS
C = 128
NCHUNK = 80
EPW = C * NCHUNK
E_PAD = NW * EPW
WB = 80
RPT = 640
RPT_LAST_CHUNKS = (N - (NS - 1) * RPT) // WB
RPT_CHUNKS = RPT // WB


def _mm_body(f_ref, wt_ref, b_ref, o_ref):
    o_ref[...] = (
        jnp.dot(f_ref[...], wt_ref[...], preferred_element_type=jnp.float32)
        + b_ref[...]
    )


def _add_body(p_ref, o_ref):
    o_ref[...] = p_ref[0] + p_ref[1]


def _bcast_lane(v, lane):
    return lax.gather(
        v, jnp.full((16, 1), lane, jnp.int32),
        lax.GatherDimensionNumbers(
            offset_dims=(), collapsed_slice_dims=(0,), start_index_map=(0,)),
        (1,), mode=lax.GatherScatterMode.PROMISE_IN_BOUNDS)


def _sc_spmm_body(support_hbm, rows_hbm, cols_hbm, vals_hbm, out_hbm,
                  cbuf, rbufa, rbufb, vbufa, vbufb, bufa, bufb, acc,
                  gsema, gsemb, esema, esemb, ssem):
    cid = lax.axis_index("c")
    sid = lax.axis_index("s")
    wid = cid * NS + sid
    zero16 = jnp.zeros((16,), jnp.float32)

    pltpu.sync_copy(cols_hbm.at[wid], cbuf)

    def zrow(g, carry):
        for j in range(D // 16):
            bufa[g, pl.ds(j * 16, 16)] = zero16
        return carry
    lax.fori_loop(0, C, zrow, 0)

    row0 = sid * RPT
    nwb = jnp.where(sid == NS - 1, RPT_LAST_CHUNKS, RPT_CHUNKS)

    def zacc(k, carry):
        pltpu.sync_copy(bufa.at[pl.ds(0, WB)], acc.at[pl.ds(row0 + k * WB, WB)])
        return carry
    lax.fori_loop(0, nwb, zacc, 0)

    plsc.subcore_barrier()

    def start_all(ci, buf, rbuf, vbuf, gsem, esem):
        pltpu.async_copy(support_hbm.at[cbuf.at[ci]], buf, gsem)
        pltpu.async_copy(rows_hbm.at[wid, ci], rbuf, esem)
        pltpu.async_copy(vals_hbm.at[wid, ci], vbuf, esem)

    def wait_all(ci, buf, rbuf, vbuf, gsem, esem):
        pltpu.make_async_copy(support_hbm.at[cbuf.at[ci]], buf, gsem).wait()
        pltpu.make_async_copy(rows_hbm.at[wid, ci], rbuf, esem).wait()
        pltpu.make_async_copy(vals_hbm.at[wid, ci], vbuf, esem).wait()

    def scale(buf, vbuf):
        def body(g, carry):
            vv = vbuf[pl.ds(g * 16, 16)]
            for i2 in range(16):
                r = g * 16 + i2
                s = _bcast_lane(vv, i2)
                for j in range(D // 16):
                    buf[r, pl.ds(j * 16, 16)] = buf[r, pl.ds(j * 16, 16)] * s
            return carry
        lax.fori_loop(0, C // 16, body, 0)

    def scatter(buf, rbuf):
        pltpu.async_copy(buf, acc.at[rbuf], ssem, add=True).wait()

    start_all(0, bufa, rbufa, vbufa, gsema, esema)

    def pair(k, carry):
        ci = 2 * k
        wait_all(ci, bufa, rbufa, vbufa, gsema, esema)
        start_all(ci + 1, bufb, rbufb, vbufb, gsemb, esemb)
        scale(bufa, vbufa)
        scatter(bufa, rbufa)
        wait_all(ci + 1, bufb, rbufb, vbufb, gsemb, esemb)

        @pl.when(k < NCHUNK // 2 - 1)
        def _():
            start_all(ci + 2, bufa, rbufa, vbufa, gsema, esema)
        scale(bufb, vbufb)
        scatter(bufb, rbufb)
        return carry
    lax.fori_loop(0, NCHUNK // 2, pair, 0)

    plsc.subcore_barrier()

    def wb(k, carry):
        sl = pl.ds(row0 + k * WB, WB)
        pltpu.sync_copy(acc.at[sl], bufa.at[pl.ds(0, WB)])
        pltpu.sync_copy(bufa.at[pl.ds(0, WB)], out_hbm.at[cid, sl])
        return carry
    lax.fori_loop(0, nwb, wb, 0)


_sc_spmm = functools.partial(
    pl.kernel,
    out_type=jax.ShapeDtypeStruct((NC, N, D), jnp.float32),
    mesh=plsc.VectorSubcoreMesh(
        core_axis_name="c", subcore_axis_name="s",
        num_cores=NC, num_subcores=NS),
    scratch_types=[
        pltpu.VMEM((NCHUNK, C), jnp.int32),
        pltpu.VMEM((C,), jnp.int32),
        pltpu.VMEM((C,), jnp.int32),
        pltpu.VMEM((C,), jnp.float32),
        pltpu.VMEM((C,), jnp.float32),
        pltpu.VMEM((C, D), jnp.float32),
        pltpu.VMEM((C, D), jnp.float32),
        pltpu.VMEM_SHARED((N, D), jnp.float32),
        pltpu.SemaphoreType.DMA,
        pltpu.SemaphoreType.DMA,
        pltpu.SemaphoreType.DMA,
        pltpu.SemaphoreType.DMA,
        pltpu.SemaphoreType.DMA,
    ],
)(_sc_spmm_body)


def kernel(laplacian_indices, laplacian_values, features, W, b):
    pad = E_PAD - E
    pad_idx = jnp.arange(pad, dtype=laplacian_indices.dtype) % N
    rows3 = jnp.concatenate([laplacian_indices[0], pad_idx]).reshape(
        NW, NCHUNK, C)
    cols3 = jnp.concatenate([laplacian_indices[1], pad_idx]).reshape(
        NW, NCHUNK, C)
    vals3 = jnp.concatenate(
        [laplacian_values, jnp.zeros((pad,), jnp.float32)]).reshape(
        NW, NCHUNK, C)
    wt = W.T
    b2 = b.reshape(1, D)

    support = pl.pallas_call(
        _mm_body,
        grid=(10,),
        in_specs=[
            pl.BlockSpec((N // 10, D), lambda i: (i, 0)),
            pl.BlockSpec((D, D), lambda i: (0, 0)),
            pl.BlockSpec((1, D), lambda i: (0, 0)),
        ],
        out_specs=pl.BlockSpec((N // 10, D), lambda i: (i, 0)),
        out_shape=jax.ShapeDtypeStruct((N, D), jnp.float32),
    )(features, wt, b2)

    partials = _sc_spmm(support, rows3, cols3, vals3)

    out = pl.pallas_call(
        _add_body,
        grid=(10,),
        in_specs=[pl.BlockSpec((NC, N // 10, D), lambda i: (0, i, 0))],
        out_specs=pl.BlockSpec((N // 10, D), lambda i: (i, 0)),
        out_shape=jax.ShapeDtypeStruct((N, D), jnp.float32),
    )(partials)
    return out


# speedup vs baseline: 2.8203x; 1.0317x over previous
import functools

import jax
import jax.numpy as jnp
from jax import lax
from jax.experimental import pallas as pl
from jax.experimental.pallas import tpu as pltpu
from jax.experimental.pallas import tpu_sc as plsc

N = 10000
E = 320000
D = 128

NC = 2
NS = 16
NW = NC * NS
EPW = E // NW
C = 128
NCHUNK = 78
NPAIR = NCHUNK // 2
REM = EPW - NCHUNK * C
WB = 80
RPT = 640
RPT_LAST_CHUNKS = (N - (NS - 1) * RPT) // WB
RPT_CHUNKS = RPT // WB


def _mm_body(f_ref, w_ref, b_ref, o_ref):
    o_ref[...] = (
        jax.lax.dot_general(
            f_ref[...], w_ref[...], (((1,), (1,)), ((), ())),
            preferred_element_type=jnp.float32)
        + b_ref[...]
    )


def _add_body(p_ref, o_ref):
    o_ref[...] = p_ref[0] + p_ref[1]


def _bcast_lane(v, lane):
    return lax.gather(
        v, jnp.full((16, 1), lane, jnp.int32),
        lax.GatherDimensionNumbers(
            offset_dims=(), collapsed_slice_dims=(0,), start_index_map=(0,)),
        (1,), mode=lax.GatherScatterMode.PROMISE_IN_BOUNDS)


def _sc_spmm_body(support_hbm, rows_hbm, cols_hbm, vals_hbm, out_hbm,
                  cbuf, rbufa, rbufb, vbufa, vbufb, bufa, bufb, acc,
                  gsema, gsemb, esema, esemb, ssem):
    cid = lax.axis_index("c")
    sid = lax.axis_index("s")
    wid = cid * NS + sid
    base_w = wid * EPW
    zero16 = jnp.zeros((16,), jnp.float32)

    pltpu.sync_copy(cols_hbm.at[pl.ds(base_w, EPW)], cbuf)

    def zrow(g, carry):
        for j in range(D // 16):
            bufa[g, pl.ds(j * 16, 16)] = zero16
        return carry
    lax.fori_loop(0, C, zrow, 0)

    row0 = sid * RPT
    nwb = jnp.where(sid == NS - 1, RPT_LAST_CHUNKS, RPT_CHUNKS)

    def zacc(k, carry):
        pltpu.sync_copy(bufa.at[pl.ds(0, WB)], acc.at[pl.ds(row0 + k * WB, WB)])
        return carry
    lax.fori_loop(0, nwb, zacc, 0)

    plsc.subcore_barrier()

    def start_all(ci, buf, rbuf, vbuf, gsem, esem):
        pltpu.async_copy(support_hbm.at[cbuf.at[pl.ds(ci * C, C)]], buf, gsem)
        pltpu.async_copy(rows_hbm.at[pl.ds(base_w + ci * C, C)], rbuf, esem)
        pltpu.async_copy(vals_hbm.at[pl.ds(base_w + ci * C, C)], vbuf, esem)

    def wait_all(ci, buf, rbuf, vbuf, gsem, esem):
        pltpu.make_async_copy(
            support_hbm.at[cbuf.at[pl.ds(ci * C, C)]], buf, gsem).wait()
        pltpu.make_async_copy(
            rows_hbm.at[pl.ds(base_w + ci * C, C)], rbuf, esem).wait()
        pltpu.make_async_copy(
            vals_hbm.at[pl.ds(base_w + ci * C, C)], vbuf, esem).wait()

    def scale(buf, vbuf, ngroup):
        def body(g, carry):
            vv = vbuf[pl.ds(g * 16, 16)]
            for i2 in range(16):
                r = g * 16 + i2
                s = _bcast_lane(vv, i2)
                for j in range(D // 16):
                    buf[r, pl.ds(j * 16, 16)] = buf[r, pl.ds(j * 16, 16)] * s
            return carry
        lax.fori_loop(0, ngroup, body, 0)

    def scatter(buf, rbuf):
        pltpu.async_copy(buf, acc.at[rbuf], ssem, add=True).wait()

    start_all(0, bufa, rbufa, vbufa, gsema, esema)

    def pair(k, carry):
        ci = 2 * k
        wait_all(ci, bufa, rbufa, vbufa, gsema, esema)
        start_all(ci + 1, bufb, rbufb, vbufb, gsemb, esemb)
        scale(bufa, vbufa, C // 16)
        scatter(bufa, rbufa)
        wait_all(ci + 1, bufb, rbufb, vbufb, gsemb, esemb)

        @pl.when(k < NPAIR - 1)
        def _():
            start_all(ci + 2, bufa, rbufa, vbufa, gsema, esema)
        scale(bufb, vbufb, C // 16)
        scatter(bufb, rbufb)
        return carry
    lax.fori_loop(0, NPAIR, pair, 0)

    rem = NCHUNK * C
    pltpu.async_copy(
        support_hbm.at[cbuf.at[pl.ds(rem, REM)]],
        bufa.at[pl.ds(0, REM)], gsema)
    pltpu.sync_copy(rows_hbm.at[pl.ds(base_w + rem, REM)],
                    rbufa.at[pl.ds(0, REM)])
    pltpu.sync_copy(vals_hbm.at[pl.ds(base_w + rem, REM)],
                    vbufa.at[pl.ds(0, REM)])
    pltpu.make_async_copy(
        support_hbm.at[cbuf.at[pl.ds(rem, REM)]],
        bufa.at[pl.ds(0, REM)], gsema).wait()
    scale(bufa, vbufa, REM // 16)
    pltpu.async_copy(bufa.at[pl.ds(0, REM)],
                     acc.at[rbufa.at[pl.ds(0, REM)]], ssem, add=True).wait()

    plsc.subcore_barrier()

    def wb(k, carry):
        sl = pl.ds(row0 + k * WB, WB)
        pltpu.sync_copy(acc.at[sl], bufa.at[pl.ds(0, WB)])
        pltpu.sync_copy(bufa.at[pl.ds(0, WB)], out_hbm.at[cid, sl])
        return carry
    lax.fori_loop(0, nwb, wb, 0)


_sc_spmm = functools.partial(
    pl.kernel,
    out_type=jax.ShapeDtypeStruct((NC, N, D), jnp.float32),
    mesh=plsc.VectorSubcoreMesh(
        core_axis_name="c", subcore_axis_name="s",
        num_cores=NC, num_subcores=NS),
    scratch_types=[
        pltpu.VMEM((EPW,), jnp.int32),
        pltpu.VMEM((C,), jnp.int32),
        pltpu.VMEM((C,), jnp.int32),
        pltpu.VMEM((C,), jnp.float32),
        pltpu.VMEM((C,), jnp.float32),
        pltpu.VMEM((C, D), jnp.float32),
        pltpu.VMEM((C, D), jnp.float32),
        pltpu.VMEM_SHARED((N, D), jnp.float32),
        pltpu.SemaphoreType.DMA,
        pltpu.SemaphoreType.DMA,
        pltpu.SemaphoreType.DMA,
        pltpu.SemaphoreType.DMA,
        pltpu.SemaphoreType.DMA,
    ],
)(_sc_spmm_body)


def kernel(laplacian_indices, laplacian_values, features, W, b):
    b2 = b.reshape(1, D)

    support = pl.pallas_call(
        _mm_body,
        grid=(5,),
        in_specs=[
            pl.BlockSpec((N // 5, D), lambda i: (i, 0)),
            pl.BlockSpec((D, D), lambda i: (0, 0)),
            pl.BlockSpec((1, D), lambda i: (0, 0)),
        ],
        out_specs=pl.BlockSpec((N // 5, D), lambda i: (i, 0)),
        out_shape=jax.ShapeDtypeStruct((N, D), jnp.float32),
    )(features, W, b2)

    partials = _sc_spmm(support, laplacian_indices[0], laplacian_indices[1],
                        laplacian_values)

    out = pl.pallas_call(
        _add_body,
        grid=(10,),
        in_specs=[pl.BlockSpec((NC, N // 10, D), lambda i: (0, i, 0))],
        out_specs=pl.BlockSpec((N // 10, D), lambda i: (i, 0)),
        out_shape=jax.ShapeDtypeStruct((N, D), jnp.float32),
    )(partials)
    return out
